# Initial kernel scaffold; baseline (speedup 1.0000x reference)
import jax
import jax.numpy as jnp
from jax.experimental import pallas as pl


def kernel(x, edge_index, W1, b1, Wg1, bg1, Wg2, bg2, W2, b2, W3, b3):
    raise NotImplementedError("write your pallas kernel here")


# speedup vs baseline: 5.9992x; 5.9992x over previous
import functools

import jax
import jax.numpy as jnp
from jax import lax
from jax.experimental import pallas as pl
from jax.experimental.pallas import tpu as pltpu
from jax.experimental.pallas import tpu_sc as plsc

N = 10000
E = 320000
D_IN = 128
H = 256
HH = 128
D_OUT = 64

NC = 2
NS = 16
CH = 128
EPAD = 327680
CPT_EDGE = EPAD // (NS * CH)
CPT_DEG = EPAD // (NC * NS * CH)
ACC_ROWS = 10240
ZR_PT = ACC_ROWS // NS
DEG_W = 128

BR = 1000
GR = N // BR


def _sc_degree(dstp, ones16, zeros16):
    mesh = plsc.VectorSubcoreMesh(core_axis_name="c", subcore_axis_name="s")

    @functools.partial(
        pl.kernel,
        out_type=(jax.ShapeDtypeStruct((N, DEG_W), jnp.float32),
                  jax.ShapeDtypeStruct((N, DEG_W), jnp.float32)),
        mesh=mesh,
        scratch_types=[
            pltpu.VMEM((CH,), jnp.int32),
            pltpu.VMEM((CH, DEG_W), jnp.float32),
            pltpu.VMEM((CH, DEG_W), jnp.float32),
            pltpu.VMEM_SHARED((ACC_ROWS, DEG_W), jnp.float32),
        ],
        interpret=False,
    )
    def k(dst_ref, ones_ref, zeros_ref, degA, degB, didx, onesbuf, stage, acc):
        c = lax.axis_index("c")
        s = lax.axis_index("s")
        pltpu.sync_copy(ones_ref, onesbuf)
        pltpu.sync_copy(zeros_ref, stage)
        for j in range(ZR_PT // CH):
            pltpu.sync_copy(stage, acc.at[pl.ds(s * ZR_PT + j * CH, CH)])
        plsc.subcore_barrier()
        w = s * NC + c

        def body(j, carry):
            base = pl.multiple_of(w * (CPT_DEG * CH) + j * CH, 8)
            pltpu.sync_copy(dst_ref.at[pl.ds(base, CH)], didx)
            pltpu.sync_copy(onesbuf, acc.at[didx], add=True)
            return carry

        lax.fori_loop(0, CPT_DEG, body, 0)
        plsc.subcore_barrier()

        def readout(out_ref):
            for j in range(ZR_PT // CH):
                rbase = pl.multiple_of(s * ZR_PT + j * CH, 8)
                full = rbase + CH <= N

                @pl.when(full)
                def _():
                    pltpu.sync_copy(acc.at[pl.ds(rbase, CH)], stage)
                    pltpu.sync_copy(stage, out_ref.at[pl.ds(rbase, CH)])

                @pl.when(jnp.logical_and(jnp.logical_not(full), rbase < N))
                def _():
                    tail = N % CH
                    pltpu.sync_copy(acc.at[pl.ds(rbase, tail)], stage.at[pl.ds(0, tail)])
                    pltpu.sync_copy(stage.at[pl.ds(0, tail)], out_ref.at[pl.ds(rbase, tail)])

        @pl.when(c == 0)
        def _():
            readout(degA)

        @pl.when(c == 1)
        def _():
            readout(degB)

    return k(dstp, ones16, zeros16)


def _sc_edge_pass(hsA, hsB, srcp, dstp, zerosH):
    mesh = plsc.VectorSubcoreMesh(core_axis_name="c", subcore_axis_name="s")

    @functools.partial(
        pl.kernel,
        out_type=(jax.ShapeDtypeStruct((N, HH), jnp.float32),
                  jax.ShapeDtypeStruct((N, HH), jnp.float32)),
        mesh=mesh,
        scratch_types=[
            pltpu.VMEM((CH,), jnp.int32),
            pltpu.VMEM((CH,), jnp.int32),
            pltpu.VMEM((CH, HH), jnp.float32),
            pltpu.VMEM((CH, HH), jnp.float32),
            pltpu.VMEM_SHARED((ACC_ROWS, HH), jnp.float32),
            pltpu.SemaphoreType.DMA,
        ],
        interpret=False,
    )
    def k(hsA_ref, hsB_ref, src_ref, dst_ref, zeros_ref, ssA, ssB,
          sidx, didx, rows, stage, acc, sem):
        c = lax.axis_index("c")
        s = lax.axis_index("s")
        pltpu.sync_copy(zeros_ref, stage)
        for j in range(ZR_PT // CH):
            pltpu.sync_copy(stage, acc.at[pl.ds(s * ZR_PT + j * CH, CH)])
        plsc.subcore_barrier()

        def run(hs_ref, ss_ref):
            def body(j, carry):
                base = pl.multiple_of(s * (CPT_EDGE * CH) + j * CH, 8)
                pltpu.sync_copy(src_ref.at[pl.ds(base, CH)], sidx)
                pltpu.sync_copy(dst_ref.at[pl.ds(base, CH)], didx)
                pltpu.async_copy(hs_ref.at[sidx], rows, sem).wait()
                pltpu.sync_copy(rows, acc.at[didx], add=True)
                return carry

            lax.fori_loop(0, CPT_EDGE, body, 0)
            plsc.subcore_barrier()
            for j in range(ZR_PT // CH):
                rbase = pl.multiple_of(s * ZR_PT + j * CH, 8)
                full = rbase + CH <= N

                @pl.when(full)
                def _():
                    pltpu.sync_copy(acc.at[pl.ds(rbase, CH)], stage)
                    pltpu.sync_copy(stage, ss_ref.at[pl.ds(rbase, CH)])

                @pl.when(jnp.logical_and(jnp.logical_not(full), rbase < N))
                def _():
                    tail = N % CH
                    pltpu.sync_copy(acc.at[pl.ds(rbase, tail)], stage.at[pl.ds(0, tail)])
                    pltpu.sync_copy(stage.at[pl.ds(0, tail)], ss_ref.at[pl.ds(rbase, tail)])

        @pl.when(c == 0)
        def _():
            run(hsA_ref, ssA)

        @pl.when(c == 1)
        def _():
            run(hsB_ref, ssB)

    return k(hsA, hsB, srcp, dstp, zerosH)


def _dinv_block(dA_ref, dB_ref):
    deg = dA_ref[:, :1] + dB_ref[:, :1] + 1.0
    return lax.rsqrt(deg)


def _tc_in_mlp(x, W1, b1):
    def body(x_ref, w_ref, b_ref, o_ref):
        o_ref[...] = jnp.maximum(
            jnp.dot(x_ref[...], w_ref[...], preferred_element_type=jnp.float32)
            + b_ref[...], 0.0)

    return pl.pallas_call(
        body,
        grid=(GR,),
        in_specs=[pl.BlockSpec((BR, D_IN), lambda i: (i, 0)),
                  pl.BlockSpec((D_IN, H), lambda i: (0, 0)),
                  pl.BlockSpec((1, H), lambda i: (0, 0))],
        out_specs=pl.BlockSpec((BR, H), lambda i: (i, 0)),
        out_shape=jax.ShapeDtypeStruct((N, H), jnp.float32),
        interpret=False,
    )(x, W1, b1.reshape(1, H))


def _tc_scale_project(h, Wg, degA, degB):
    def body(h_ref, w_ref, dA_ref, dB_ref, oA, oB):
        dinv = _dinv_block(dA_ref, dB_ref)
        hw = jnp.dot(h_ref[...], w_ref[...], preferred_element_type=jnp.float32)
        hs = hw * dinv
        oA[...] = hs[:, :HH]
        oB[...] = hs[:, HH:]

    return pl.pallas_call(
        body,
        grid=(GR,),
        in_specs=[pl.BlockSpec((BR, H), lambda i: (i, 0)),
                  pl.BlockSpec((H, H), lambda i: (0, 0)),
                  pl.BlockSpec((BR, DEG_W), lambda i: (i, 0)),
                  pl.BlockSpec((BR, DEG_W), lambda i: (i, 0))],
        out_specs=[pl.BlockSpec((BR, HH), lambda i: (i, 0)),
                   pl.BlockSpec((BR, HH), lambda i: (i, 0))],
        out_shape=[jax.ShapeDtypeStruct((N, HH), jnp.float32),
                   jax.ShapeDtypeStruct((N, HH), jnp.float32)],
        interpret=False,
    )(h, Wg, degA, degB)


def _tc_gcn_finish_project(ssA, ssB, hsA, hsB, degA, degB, bg, Wg2):
    def body(ssA_ref, ssB_ref, hsA_ref, hsB_ref, dA_ref, dB_ref, bg_ref,
             w_ref, oA, oB):
        dinv = _dinv_block(dA_ref, dB_ref)
        h = jnp.concatenate(
            [ssA_ref[...] + hsA_ref[...], ssB_ref[...] + hsB_ref[...]], axis=1)
        h = jnp.maximum(h * dinv + bg_ref[...], 0.0)
        hw = jnp.dot(h, w_ref[...], preferred_element_type=jnp.float32)
        hs = hw * dinv
        oA[...] = hs[:, :HH]
        oB[...] = hs[:, HH:]

    return pl.pallas_call(
        body,
        grid=(GR,),
        in_specs=[pl.BlockSpec((BR, HH), lambda i: (i, 0)),
                  pl.BlockSpec((BR, HH), lambda i: (i, 0)),
                  pl.BlockSpec((BR, HH), lambda i: (i, 0)),
                  pl.BlockSpec((BR, HH), lambda i: (i, 0)),
                  pl.BlockSpec((BR, DEG_W), lambda i: (i, 0)),
                  pl.BlockSpec((BR, DEG_W), lambda i: (i, 0)),
                  pl.BlockSpec((1, H), lambda i: (0, 0)),
                  pl.BlockSpec((H, H), lambda i: (0, 0))],
        out_specs=[pl.BlockSpec((BR, HH), lambda i: (i, 0)),
                   pl.BlockSpec((BR, HH), lambda i: (i, 0))],
        out_shape=[jax.ShapeDtypeStruct((N, HH), jnp.float32),
                   jax.ShapeDtypeStruct((N, HH), jnp.float32)],
        interpret=False,
    )(ssA, ssB, hsA, hsB, degA, degB, bg.reshape(1, H), Wg2)


def _tc_final(ssA, ssB, hsA, hsB, degA, degB, bg2, W2, b2, W3, b3):
    def body(ssA_ref, ssB_ref, hsA_ref, hsB_ref, dA_ref, dB_ref, bg2_ref,
             w2_ref, b2_ref, w3_ref, b3_ref, o_ref):
        dinv = _dinv_block(dA_ref, dB_ref)
        h = jnp.concatenate(
            [ssA_ref[...] + hsA_ref[...], ssB_ref[...] + hsB_ref[...]], axis=1)
        h = jnp.maximum(h * dinv + bg2_ref[...], 0.0)
        h = jnp.maximum(
            jnp.dot(h, w2_ref[...], preferred_element_type=jnp.float32)
            + b2_ref[...], 0.0)
        z = jnp.dot(h, w3_ref[...], preferred_element_type=jnp.float32) + b3_ref[...]
        z = z - jnp.max(z, axis=1, keepdims=True)
        ez = jnp.exp(z)
        o_ref[...] = ez / jnp.sum(ez, axis=1, keepdims=True)

    return pl.pallas_call(
        body,
        grid=(GR,),
        in_specs=[pl.BlockSpec((BR, HH), lambda i: (i, 0)),
                  pl.BlockSpec((BR, HH), lambda i: (i, 0)),
                  pl.BlockSpec((BR, HH), lambda i: (i, 0)),
                  pl.BlockSpec((BR, HH), lambda i: (i, 0)),
                  pl.BlockSpec((BR, DEG_W), lambda i: (i, 0)),
                  pl.BlockSpec((BR, DEG_W), lambda i: (i, 0)),
                  pl.BlockSpec((1, H), lambda i: (0, 0)),
                  pl.BlockSpec((H, H), lambda i: (0, 0)),
                  pl.BlockSpec((1, H), lambda i: (0, 0)),
                  pl.BlockSpec((H, D_OUT), lambda i: (0, 0)),
                  pl.BlockSpec((1, D_OUT), lambda i: (0, 0))],
        out_specs=pl.BlockSpec((BR, D_OUT), lambda i: (i, 0)),
        out_shape=jax.ShapeDtypeStruct((N, D_OUT), jnp.float32),
        interpret=False,
    )(ssA, ssB, hsA, hsB, degA, degB, bg2.reshape(1, H), W2,
      b2.reshape(1, H), W3, b3.reshape(1, D_OUT))


def kernel(x, edge_index, W1, b1, Wg1, bg1, Wg2, bg2, W2, b2, W3, b3):
    src = edge_index[0].astype(jnp.int32)
    dst = edge_index[1].astype(jnp.int32)
    srcp = jnp.concatenate([src, jnp.zeros((EPAD - E,), jnp.int32)])
    dstp = jnp.concatenate([dst, jnp.full((EPAD - E,), N, jnp.int32)])
    ones16 = jnp.ones((CH, DEG_W), jnp.float32)
    zeros16 = jnp.zeros((CH, DEG_W), jnp.float32)
    zerosH = jnp.zeros((CH, HH), jnp.float32)

    degA, degB = _sc_degree(dstp, ones16, zeros16)
    h0 = _tc_in_mlp(x, W1, b1)
    hs1A, hs1B = _tc_scale_project(h0, Wg1, degA, degB)
    ss1A, ss1B = _sc_edge_pass(hs1A, hs1B, srcp, dstp, zerosH)
    hs2A, hs2B = _tc_gcn_finish_project(ss1A, ss1B, hs1A, hs1B, degA, degB, bg1, Wg2)
    ss2A, ss2B = _sc_edge_pass(hs2A, hs2B, srcp, dstp, zerosH)
    return _tc_final(ss2A, ss2B, hs2A, hs2B, degA, degB, bg2, W2, b2, W3, b3)


# speedup vs baseline: 7.4666x; 1.2446x over previous
import functools

import jax
import jax.numpy as jnp
from jax import lax
from jax.experimental import pallas as pl
from jax.experimental.pallas import tpu as pltpu
from jax.experimental.pallas import tpu_sc as plsc

N = 10000
E = 320000
D_IN = 128
H = 256
HH = 128
D_OUT = 64

NC = 2
NS = 16
CH = 128
EPAD = 327680
CPT_EDGE = EPAD // (NS * CH)
CPT_DEG = EPAD // (NC * NS * CH)
GW = 2
N_GROUPS = EPAD // (GW * CH)
GPT_EDGE = CPT_EDGE // GW
GPT_DEG = CPT_DEG // GW
ACC_ROWS = 10240
ZR_PT = ACC_ROWS // NS
DEG_W = 128

BR = 1000
GR = N // BR


def _sc_degree(dstp, ones128, zeros128):
    mesh = plsc.VectorSubcoreMesh(core_axis_name="c", subcore_axis_name="s")

    @functools.partial(
        pl.kernel,
        out_type=(jax.ShapeDtypeStruct((N, DEG_W), jnp.float32),
                  jax.ShapeDtypeStruct((N, DEG_W), jnp.float32)),
        mesh=mesh,
        scratch_types=[
            pltpu.VMEM((CPT_DEG, CH), jnp.int32),
            pltpu.VMEM((CH, DEG_W), jnp.float32),
            pltpu.VMEM((CH, DEG_W), jnp.float32),
            pltpu.VMEM_SHARED((ACC_ROWS, DEG_W), jnp.float32),
            pltpu.SemaphoreType.DMA,
            pltpu.SemaphoreType.DMA,
        ],
        interpret=False,
    )
    def k(dst_ref, ones_ref, zeros_ref, degA, degB,
          didx2, onesbuf, stage, acc, bsem, ssem):
        c = lax.axis_index("c")
        s = lax.axis_index("s")
        pltpu.sync_copy(ones_ref, onesbuf)
        pltpu.sync_copy(zeros_ref, stage)
        for j in range(ZR_PT // CH):
            pltpu.sync_copy(stage, acc.at[pl.ds(s * ZR_PT + j * CH, CH)])
        w = s * NC + c
        bds = [pltpu.async_copy(
                   dst_ref.at[pl.ds(pl.multiple_of(w * (CPT_DEG * CH) + j * CH, 8), CH)],
                   didx2.at[j], bsem)
               for j in range(CPT_DEG)]
        for d in bds:
            d.wait()
        plsc.subcore_barrier()

        def body(g, carry):
            sds = [pltpu.async_copy(onesbuf, acc.at[didx2.at[g * GW + u]],
                                    ssem, add=True)
                   for u in range(GW)]
            for d in sds:
                d.wait()
            return carry

        lax.fori_loop(0, GPT_DEG, body, 0)
        plsc.subcore_barrier()

        def readout(out_ref):
            for j in range(ZR_PT // CH):
                rbase = pl.multiple_of(s * ZR_PT + j * CH, 8)
                full = rbase + CH <= N

                @pl.when(full)
                def _():
                    pltpu.sync_copy(acc.at[pl.ds(rbase, CH)], stage)
                    pltpu.sync_copy(stage, out_ref.at[pl.ds(rbase, CH)])

                @pl.when(jnp.logical_and(jnp.logical_not(full), rbase < N))
                def _():
                    tail = N % CH
                    pltpu.sync_copy(acc.at[pl.ds(rbase, tail)], stage.at[pl.ds(0, tail)])
                    pltpu.sync_copy(stage.at[pl.ds(0, tail)], out_ref.at[pl.ds(rbase, tail)])

        @pl.when(c == 0)
        def _():
            readout(degA)

        @pl.when(c == 1)
        def _():
            readout(degB)

    return k(dstp, ones128, zeros128)


def _sc_edge_pass(hsA, hsB, srcp, dstp, zerosH):
    mesh = plsc.VectorSubcoreMesh(core_axis_name="c", subcore_axis_name="s")

    @functools.partial(
        pl.kernel,
        out_type=(jax.ShapeDtypeStruct((N, HH), jnp.float32),
                  jax.ShapeDtypeStruct((N, HH), jnp.float32)),
        mesh=mesh,
        scratch_types=[
            pltpu.VMEM((GW, CH), jnp.int32),
            pltpu.VMEM((GW, CH), jnp.int32),
            pltpu.VMEM((GW, CH, HH), jnp.float32),
            pltpu.VMEM_SHARED((ACC_ROWS, HH), jnp.float32),
            pltpu.SemaphoreType.DMA,
            pltpu.SemaphoreType.DMA,
            pltpu.SemaphoreType.DMA,
        ],
        interpret=False,
    )
    def k(hsA_ref, hsB_ref, src_ref, dst_ref, zeros_ref, ssA, ssB,
          sidx, didx, rows, acc, bsem, gsem, ssem):
        c = lax.axis_index("c")
        s = lax.axis_index("s")
        stage = rows.at[0]
        pltpu.sync_copy(zeros_ref, stage)
        for j in range(ZR_PT // CH):
            pltpu.sync_copy(stage, acc.at[pl.ds(s * ZR_PT + j * CH, CH)])
        plsc.subcore_barrier()

        def run(hs_ref, ss_ref):
            def body(g, carry):
                bds = []
                for u in range(GW):
                    base = pl.multiple_of(
                        s * (CPT_EDGE * CH) + (g * GW + u) * CH, 8)
                    bds.append(pltpu.async_copy(src_ref.at[pl.ds(base, CH)],
                                                sidx.at[u], bsem))
                    bds.append(pltpu.async_copy(dst_ref.at[pl.ds(base, CH)],
                                                didx.at[u], bsem))
                for d in bds:
                    d.wait()
                gds = [pltpu.async_copy(hs_ref.at[sidx.at[u]],
                                        rows.at[u], gsem)
                       for u in range(GW)]
                sds = []
                for u in range(GW):
                    gds[u].wait()
                    sds.append(pltpu.async_copy(rows.at[u],
                                                acc.at[didx.at[u]],
                                                ssem, add=True))
                for d in sds:
                    d.wait()
                return carry

            lax.fori_loop(0, GPT_EDGE, body, 0)
            plsc.subcore_barrier()
            for j in range(ZR_PT // CH):
                rbase = pl.multiple_of(s * ZR_PT + j * CH, 8)
                full = rbase + CH <= N

                @pl.when(full)
                def _():
                    pltpu.sync_copy(acc.at[pl.ds(rbase, CH)], stage)
                    pltpu.sync_copy(stage, ss_ref.at[pl.ds(rbase, CH)])

                @pl.when(jnp.logical_and(jnp.logical_not(full), rbase < N))
                def _():
                    tail = N % CH
                    pltpu.sync_copy(acc.at[pl.ds(rbase, tail)],
                                    stage.at[pl.ds(0, tail)])
                    pltpu.sync_copy(stage.at[pl.ds(0, tail)],
                                    ss_ref.at[pl.ds(rbase, tail)])

        @pl.when(c == 0)
        def _():
            run(hsA_ref, ssA)

        @pl.when(c == 1)
        def _():
            run(hsB_ref, ssB)

    return k(hsA, hsB, srcp, dstp, zerosH)


def _dinv_block(dA_ref, dB_ref):
    deg = dA_ref[:, :1] + dB_ref[:, :1] + 1.0
    return lax.rsqrt(deg)


def _tc_in_mlp(x, W1, b1):
    def body(x_ref, w_ref, b_ref, o_ref):
        o_ref[...] = jnp.maximum(
            jnp.dot(x_ref[...], w_ref[...], preferred_element_type=jnp.float32)
            + b_ref[...], 0.0)

    return pl.pallas_call(
        body,
        grid=(GR,),
        in_specs=[pl.BlockSpec((BR, D_IN), lambda i: (i, 0)),
                  pl.BlockSpec((D_IN, H), lambda i: (0, 0)),
                  pl.BlockSpec((1, H), lambda i: (0, 0))],
        out_specs=pl.BlockSpec((BR, H), lambda i: (i, 0)),
        out_shape=jax.ShapeDtypeStruct((N, H), jnp.float32),
        interpret=False,
    )(x, W1, b1.reshape(1, H))


def _tc_scale_project(h, Wg, degA, degB):
    def body(h_ref, w_ref, dA_ref, dB_ref, oA, oB):
        dinv = _dinv_block(dA_ref, dB_ref)
        hw = jnp.dot(h_ref[...], w_ref[...], preferred_element_type=jnp.float32)
        hs = hw * dinv
        oA[...] = hs[:, :HH]
        oB[...] = hs[:, HH:]

    return pl.pallas_call(
        body,
        grid=(GR,),
        in_specs=[pl.BlockSpec((BR, H), lambda i: (i, 0)),
                  pl.BlockSpec((H, H), lambda i: (0, 0)),
                  pl.BlockSpec((BR, DEG_W), lambda i: (i, 0)),
                  pl.BlockSpec((BR, DEG_W), lambda i: (i, 0))],
        out_specs=[pl.BlockSpec((BR, HH), lambda i: (i, 0)),
                   pl.BlockSpec((BR, HH), lambda i: (i, 0))],
        out_shape=[jax.ShapeDtypeStruct((N, HH), jnp.float32),
                   jax.ShapeDtypeStruct((N, HH), jnp.float32)],
        interpret=False,
    )(h, Wg, degA, degB)


def _tc_gcn_finish_project(ssA, ssB, hsA, hsB, degA, degB, bg, Wg2):
    def body(ssA_ref, ssB_ref, hsA_ref, hsB_ref, dA_ref, dB_ref, bg_ref,
             w_ref, oA, oB):
        dinv = _dinv_block(dA_ref, dB_ref)
        h = jnp.concatenate(
            [ssA_ref[...] + hsA_ref[...], ssB_ref[...] + hsB_ref[...]], axis=1)
        h = jnp.maximum(h * dinv + bg_ref[...], 0.0)
        hw = jnp.dot(h, w_ref[...], preferred_element_type=jnp.float32)
        hs = hw * dinv
        oA[...] = hs[:, :HH]
        oB[...] = hs[:, HH:]

    return pl.pallas_call(
        body,
        grid=(GR,),
        in_specs=[pl.BlockSpec((BR, HH), lambda i: (i, 0)),
                  pl.BlockSpec((BR, HH), lambda i: (i, 0)),
                  pl.BlockSpec((BR, HH), lambda i: (i, 0)),
                  pl.BlockSpec((BR, HH), lambda i: (i, 0)),
                  pl.BlockSpec((BR, DEG_W), lambda i: (i, 0)),
                  pl.BlockSpec((BR, DEG_W), lambda i: (i, 0)),
                  pl.BlockSpec((1, H), lambda i: (0, 0)),
                  pl.BlockSpec((H, H), lambda i: (0, 0))],
        out_specs=[pl.BlockSpec((BR, HH), lambda i: (i, 0)),
                   pl.BlockSpec((BR, HH), lambda i: (i, 0))],
        out_shape=[jax.ShapeDtypeStruct((N, HH), jnp.float32),
                   jax.ShapeDtypeStruct((N, HH), jnp.float32)],
        interpret=False,
    )(ssA, ssB, hsA, hsB, degA, degB, bg.reshape(1, H), Wg2)


def _tc_final(ssA, ssB, hsA, hsB, degA, degB, bg2, W2, b2, W3, b3):
    def body(ssA_ref, ssB_ref, hsA_ref, hsB_ref, dA_ref, dB_ref, bg2_ref,
             w2_ref, b2_ref, w3_ref, b3_ref, o_ref):
        dinv = _dinv_block(dA_ref, dB_ref)
        h = jnp.concatenate(
            [ssA_ref[...] + hsA_ref[...], ssB_ref[...] + hsB_ref[...]], axis=1)
        h = jnp.maximum(h * dinv + bg2_ref[...], 0.0)
        h = jnp.maximum(
            jnp.dot(h, w2_ref[...], preferred_element_type=jnp.float32)
            + b2_ref[...], 0.0)
        z = jnp.dot(h, w3_ref[...], preferred_element_type=jnp.float32) + b3_ref[...]
        z = z - jnp.max(z, axis=1, keepdims=True)
        ez = jnp.exp(z)
        o_ref[...] = ez / jnp.sum(ez, axis=1, keepdims=True)

    return pl.pallas_call(
        body,
        grid=(GR,),
        in_specs=[pl.BlockSpec((BR, HH), lambda i: (i, 0)),
                  pl.BlockSpec((BR, HH), lambda i: (i, 0)),
                  pl.BlockSpec((BR, HH), lambda i: (i, 0)),
                  pl.BlockSpec((BR, HH), lambda i: (i, 0)),
                  pl.BlockSpec((BR, DEG_W), lambda i: (i, 0)),
                  pl.BlockSpec((BR, DEG_W), lambda i: (i, 0)),
                  pl.BlockSpec((1, H), lambda i: (0, 0)),
                  pl.BlockSpec((H, H), lambda i: (0, 0)),
                  pl.BlockSpec((1, H), lambda i: (0, 0)),
                  pl.BlockSpec((H, D_OUT), lambda i: (0, 0)),
                  pl.BlockSpec((1, D_OUT), lambda i: (0, 0))],
        out_specs=pl.BlockSpec((BR, D_OUT), lambda i: (i, 0)),
        out_shape=jax.ShapeDtypeStruct((N, D_OUT), jnp.float32),
        interpret=False,
    )(ssA, ssB, hsA, hsB, degA, degB, bg2.reshape(1, H), W2,
      b2.reshape(1, H), W3, b3.reshape(1, D_OUT))


def kernel(x, edge_index, W1, b1, Wg1, bg1, Wg2, bg2, W2, b2, W3, b3):
    src = edge_index[0].astype(jnp.int32)
    dst = edge_index[1].astype(jnp.int32)
    srcp = jnp.concatenate([src, jnp.zeros((EPAD - E,), jnp.int32)])
    dstp = jnp.concatenate([dst, jnp.full((EPAD - E,), N, jnp.int32)])
    ones128 = jnp.ones((CH, DEG_W), jnp.float32)
    zeros128 = jnp.zeros((CH, DEG_W), jnp.float32)
    zerosH = jnp.zeros((CH, HH), jnp.float32)

    degA, degB = _sc_degree(dstp, ones128, zeros128)
    h0 = _tc_in_mlp(x, W1, b1)
    hs1A, hs1B = _tc_scale_project(h0, Wg1, degA, degB)
    ss1A, ss1B = _sc_edge_pass(hs1A, hs1B, srcp, dstp, zerosH)
    hs2A, hs2B = _tc_gcn_finish_project(ss1A, ss1B, hs1A, hs1B, degA, degB, bg1, Wg2)
    ss2A, ss2B = _sc_edge_pass(hs2A, hs2B, srcp, dstp, zerosH)
    return _tc_final(ss2A, ss2B, hs2A, hs2B, degA, degB, bg2, W2, b2, W3, b3)


# speedup vs baseline: 7.8068x; 1.0456x over previous
import functools

import jax
import jax.numpy as jnp
from jax import lax
from jax.experimental import pallas as pl
from jax.experimental.pallas import tpu as pltpu
from jax.experimental.pallas import tpu_sc as plsc

N = 10000
E = 320000
D_IN = 128
H = 256
HH = 128
D_OUT = 64

NC = 2
NS = 16
CH = 128
EPAD = 327680
CPT_EDGE = EPAD // (NS * CH)
CPT_DEG = EPAD // (NC * NS * CH)
GW = 2
N_GROUPS = EPAD // (GW * CH)
GPT_EDGE = CPT_EDGE // GW
GPT_DEG = CPT_DEG // GW
ACC_ROWS = 10240
ZR_PT = ACC_ROWS // NS
DEG_W = 128

BR = 1000
GR = N // BR


def _sc_degree(dstp, ones128, zeros128):
    mesh = plsc.VectorSubcoreMesh(core_axis_name="c", subcore_axis_name="s")

    @functools.partial(
        pl.kernel,
        out_type=(jax.ShapeDtypeStruct((N, DEG_W), jnp.float32),
                  jax.ShapeDtypeStruct((N, DEG_W), jnp.float32)),
        mesh=mesh,
        scratch_types=[
            pltpu.VMEM((CPT_DEG, CH), jnp.int32),
            pltpu.VMEM((CH, DEG_W), jnp.float32),
            pltpu.VMEM((CH, DEG_W), jnp.float32),
            pltpu.VMEM_SHARED((ACC_ROWS, DEG_W), jnp.float32),
            pltpu.SemaphoreType.DMA,
            pltpu.SemaphoreType.DMA,
        ],
        interpret=False,
    )
    def k(dst_ref, ones_ref, zeros_ref, degA, degB,
          didx2, onesbuf, stage, acc, bsem, ssem):
        c = lax.axis_index("c")
        s = lax.axis_index("s")
        pltpu.sync_copy(ones_ref, onesbuf)
        pltpu.sync_copy(zeros_ref, stage)
        for j in range(ZR_PT // CH):
            pltpu.sync_copy(stage, acc.at[pl.ds(s * ZR_PT + j * CH, CH)])
        w = s * NC + c
        bds = [pltpu.async_copy(
                   dst_ref.at[pl.ds(pl.multiple_of(w * (CPT_DEG * CH) + j * CH, 8), CH)],
                   didx2.at[j], bsem)
               for j in range(CPT_DEG)]
        for d in bds:
            d.wait()
        plsc.subcore_barrier()

        def body(g, carry):
            sds = [pltpu.async_copy(onesbuf, acc.at[didx2.at[g * GW + u]],
                                    ssem, add=True)
                   for u in range(GW)]
            for d in sds:
                d.wait()
            return carry

        lax.fori_loop(0, GPT_DEG, body, 0)
        plsc.subcore_barrier()

        def readout(out_ref):
            for j in range(ZR_PT // CH):
                rbase = pl.multiple_of(s * ZR_PT + j * CH, 8)
                full = rbase + CH <= N

                @pl.when(full)
                def _():
                    pltpu.sync_copy(acc.at[pl.ds(rbase, CH)], stage)
                    pltpu.sync_copy(stage, out_ref.at[pl.ds(rbase, CH)])

                @pl.when(jnp.logical_and(jnp.logical_not(full), rbase < N))
                def _():
                    tail = N % CH
                    pltpu.sync_copy(acc.at[pl.ds(rbase, tail)], stage.at[pl.ds(0, tail)])
                    pltpu.sync_copy(stage.at[pl.ds(0, tail)], out_ref.at[pl.ds(rbase, tail)])

        @pl.when(c == 0)
        def _():
            readout(degA)

        @pl.when(c == 1)
        def _():
            readout(degB)

    return k(dstp, ones128, zeros128)


def _sc_edge_pass(hsA, hsB, srcp, dstp, zerosH):
    mesh = plsc.VectorSubcoreMesh(core_axis_name="c", subcore_axis_name="s")

    @functools.partial(
        pl.kernel,
        out_type=(jax.ShapeDtypeStruct((N, HH), jnp.float32),
                  jax.ShapeDtypeStruct((N, HH), jnp.float32)),
        mesh=mesh,
        scratch_types=[
            pltpu.VMEM((GW, CH), jnp.int32),
            pltpu.VMEM((GW, CH), jnp.int32),
            pltpu.VMEM((GW, CH), jnp.int32),
            pltpu.VMEM((GW, CH), jnp.int32),
            pltpu.VMEM((GW, CH, HH), jnp.float32),
            pltpu.VMEM_SHARED((ACC_ROWS, HH), jnp.float32),
            pltpu.SemaphoreType.DMA,
            pltpu.SemaphoreType.DMA,
            pltpu.SemaphoreType.DMA,
            pltpu.SemaphoreType.DMA,
        ],
        interpret=False,
    )
    def k(hsA_ref, hsB_ref, src_ref, dst_ref, zeros_ref, ssA, ssB,
          sidxA, didxA, sidxB, didxB, rows, acc, bsemA, bsemB, gsem, ssem):
        c = lax.axis_index("c")
        s = lax.axis_index("s")
        stage = rows.at[0]
        pltpu.sync_copy(zeros_ref, stage)
        for j in range(ZR_PT // CH):
            pltpu.sync_copy(stage, acc.at[pl.ds(s * ZR_PT + j * CH, CH)])
        plsc.subcore_barrier()

        def fire_idx(bank_s, bank_d, g, sem):
            for u in range(GW):
                base = pl.multiple_of(
                    s * (CPT_EDGE * CH) + (g * GW + u) * CH, 8)
                pltpu.async_copy(src_ref.at[pl.ds(base, CH)], bank_s.at[u], sem)
                pltpu.async_copy(dst_ref.at[pl.ds(base, CH)], bank_d.at[u], sem)

        def drain_idx(bank_s, bank_d, sem):
            for u in range(GW):
                pltpu.make_async_copy(src_ref.at[pl.ds(0, CH)],
                                      bank_s.at[u], sem).wait()
                pltpu.make_async_copy(dst_ref.at[pl.ds(0, CH)],
                                      bank_d.at[u], sem).wait()

        def run(hs_ref, ss_ref):
            def data_phase(bank_s, bank_d):
                gds = [pltpu.async_copy(hs_ref.at[bank_s.at[u]],
                                        rows.at[u], gsem)
                       for u in range(GW)]
                sds = []
                for u in range(GW):
                    gds[u].wait()
                    sds.append(pltpu.async_copy(rows.at[u],
                                                acc.at[bank_d.at[u]],
                                                ssem, add=True))
                for d in sds:
                    d.wait()

            fire_idx(sidxA, didxA, 0, bsemA)

            def body(r, carry):
                gB = 2 * r + 1
                fire_idx(sidxB, didxB, gB, bsemB)
                drain_idx(sidxA, didxA, bsemA)
                data_phase(sidxA, didxA)

                @pl.when(2 * r + 2 < GPT_EDGE)
                def _():
                    fire_idx(sidxA, didxA, 2 * r + 2, bsemA)

                drain_idx(sidxB, didxB, bsemB)
                data_phase(sidxB, didxB)
                return carry

            lax.fori_loop(0, GPT_EDGE // 2, body, 0)
            plsc.subcore_barrier()
            for j in range(ZR_PT // CH):
                rbase = pl.multiple_of(s * ZR_PT + j * CH, 8)
                full = rbase + CH <= N

                @pl.when(full)
                def _():
                    pltpu.sync_copy(acc.at[pl.ds(rbase, CH)], stage)
                    pltpu.sync_copy(stage, ss_ref.at[pl.ds(rbase, CH)])

                @pl.when(jnp.logical_and(jnp.logical_not(full), rbase < N))
                def _():
                    tail = N % CH
                    pltpu.sync_copy(acc.at[pl.ds(rbase, tail)],
                                    stage.at[pl.ds(0, tail)])
                    pltpu.sync_copy(stage.at[pl.ds(0, tail)],
                                    ss_ref.at[pl.ds(rbase, tail)])

        @pl.when(c == 0)
        def _():
            run(hsA_ref, ssA)

        @pl.when(c == 1)
        def _():
            run(hsB_ref, ssB)

    return k(hsA, hsB, srcp, dstp, zerosH)


def _dinv_block(dA_ref, dB_ref):
    deg = dA_ref[:, :1] + dB_ref[:, :1] + 1.0
    return lax.rsqrt(deg)


def _tc_in_mlp(x, W1, b1):
    def body(x_ref, w_ref, b_ref, o_ref):
        o_ref[...] = jnp.maximum(
            jnp.dot(x_ref[...], w_ref[...], preferred_element_type=jnp.float32)
            + b_ref[...], 0.0)

    return pl.pallas_call(
        body,
        grid=(GR,),
        in_specs=[pl.BlockSpec((BR, D_IN), lambda i: (i, 0)),
                  pl.BlockSpec((D_IN, H), lambda i: (0, 0)),
                  pl.BlockSpec((1, H), lambda i: (0, 0))],
        out_specs=pl.BlockSpec((BR, H), lambda i: (i, 0)),
        out_shape=jax.ShapeDtypeStruct((N, H), jnp.float32),
        interpret=False,
    )(x, W1, b1.reshape(1, H))


def _tc_scale_project(h, Wg, degA, degB):
    def body(h_ref, w_ref, dA_ref, dB_ref, oA, oB):
        dinv = _dinv_block(dA_ref, dB_ref)
        hw = jnp.dot(h_ref[...], w_ref[...], preferred_element_type=jnp.float32)
        hs = hw * dinv
        oA[...] = hs[:, :HH]
        oB[...] = hs[:, HH:]

    return pl.pallas_call(
        body,
        grid=(GR,),
        in_specs=[pl.BlockSpec((BR, H), lambda i: (i, 0)),
                  pl.BlockSpec((H, H), lambda i: (0, 0)),
                  pl.BlockSpec((BR, DEG_W), lambda i: (i, 0)),
                  pl.BlockSpec((BR, DEG_W), lambda i: (i, 0))],
        out_specs=[pl.BlockSpec((BR, HH), lambda i: (i, 0)),
                   pl.BlockSpec((BR, HH), lambda i: (i, 0))],
        out_shape=[jax.ShapeDtypeStruct((N, HH), jnp.float32),
                   jax.ShapeDtypeStruct((N, HH), jnp.float32)],
        interpret=False,
    )(h, Wg, degA, degB)


def _tc_gcn_finish_project(ssA, ssB, hsA, hsB, degA, degB, bg, Wg2):
    def body(ssA_ref, ssB_ref, hsA_ref, hsB_ref, dA_ref, dB_ref, bg_ref,
             w_ref, oA, oB):
        dinv = _dinv_block(dA_ref, dB_ref)
        h = jnp.concatenate(
            [ssA_ref[...] + hsA_ref[...], ssB_ref[...] + hsB_ref[...]], axis=1)
        h = jnp.maximum(h * dinv + bg_ref[...], 0.0)
        hw = jnp.dot(h, w_ref[...], preferred_element_type=jnp.float32)
        hs = hw * dinv
        oA[...] = hs[:, :HH]
        oB[...] = hs[:, HH:]

    return pl.pallas_call(
        body,
        grid=(GR,),
        in_specs=[pl.BlockSpec((BR, HH), lambda i: (i, 0)),
                  pl.BlockSpec((BR, HH), lambda i: (i, 0)),
                  pl.BlockSpec((BR, HH), lambda i: (i, 0)),
                  pl.BlockSpec((BR, HH), lambda i: (i, 0)),
                  pl.BlockSpec((BR, DEG_W), lambda i: (i, 0)),
                  pl.BlockSpec((BR, DEG_W), lambda i: (i, 0)),
                  pl.BlockSpec((1, H), lambda i: (0, 0)),
                  pl.BlockSpec((H, H), lambda i: (0, 0))],
        out_specs=[pl.BlockSpec((BR, HH), lambda i: (i, 0)),
                   pl.BlockSpec((BR, HH), lambda i: (i, 0))],
        out_shape=[jax.ShapeDtypeStruct((N, HH), jnp.float32),
                   jax.ShapeDtypeStruct((N, HH), jnp.float32)],
        interpret=False,
    )(ssA, ssB, hsA, hsB, degA, degB, bg.reshape(1, H), Wg2)


def _tc_final(ssA, ssB, hsA, hsB, degA, degB, bg2, W2, b2, W3, b3):
    def body(ssA_ref, ssB_ref, hsA_ref, hsB_ref, dA_ref, dB_ref, bg2_ref,
             w2_ref, b2_ref, w3_ref, b3_ref, o_ref):
        dinv = _dinv_block(dA_ref, dB_ref)
        h = jnp.concatenate(
            [ssA_ref[...] + hsA_ref[...], ssB_ref[...] + hsB_ref[...]], axis=1)
        h = jnp.maximum(h * dinv + bg2_ref[...], 0.0)
        h = jnp.maximum(
            jnp.dot(h, w2_ref[...], preferred_element_type=jnp.float32)
            + b2_ref[...], 0.0)
        z = jnp.dot(h, w3_ref[...], preferred_element_type=jnp.float32) + b3_ref[...]
        z = z - jnp.max(z, axis=1, keepdims=True)
        ez = jnp.exp(z)
        o_ref[...] = ez / jnp.sum(ez, axis=1, keepdims=True)

    return pl.pallas_call(
        body,
        grid=(GR,),
        in_specs=[pl.BlockSpec((BR, HH), lambda i: (i, 0)),
                  pl.BlockSpec((BR, HH), lambda i: (i, 0)),
                  pl.BlockSpec((BR, HH), lambda i: (i, 0)),
                  pl.BlockSpec((BR, HH), lambda i: (i, 0)),
                  pl.BlockSpec((BR, DEG_W), lambda i: (i, 0)),
                  pl.BlockSpec((BR, DEG_W), lambda i: (i, 0)),
                  pl.BlockSpec((1, H), lambda i: (0, 0)),
                  pl.BlockSpec((H, H), lambda i: (0, 0)),
                  pl.BlockSpec((1, H), lambda i: (0, 0)),
                  pl.BlockSpec((H, D_OUT), lambda i: (0, 0)),
                  pl.BlockSpec((1, D_OUT), lambda i: (0, 0))],
        out_specs=pl.BlockSpec((BR, D_OUT), lambda i: (i, 0)),
        out_shape=jax.ShapeDtypeStruct((N, D_OUT), jnp.float32),
        interpret=False,
    )(ssA, ssB, hsA, hsB, degA, degB, bg2.reshape(1, H), W2,
      b2.reshape(1, H), W3, b3.reshape(1, D_OUT))


def kernel(x, edge_index, W1, b1, Wg1, bg1, Wg2, bg2, W2, b2, W3, b3):
    src = edge_index[0].astype(jnp.int32)
    dst = edge_index[1].astype(jnp.int32)
    srcp = jnp.concatenate([src, jnp.zeros((EPAD - E,), jnp.int32)])
    dstp = jnp.concatenate([dst, jnp.full((EPAD - E,), N, jnp.int32)])
    ones128 = jnp.ones((CH, DEG_W), jnp.float32)
    zeros128 = jnp.zeros((CH, DEG_W), jnp.float32)
    zerosH = jnp.zeros((CH, HH), jnp.float32)

    degA, degB = _sc_degree(dstp, ones128, zeros128)
    h0 = _tc_in_mlp(x, W1, b1)
    hs1A, hs1B = _tc_scale_project(h0, Wg1, degA, degB)
    ss1A, ss1B = _sc_edge_pass(hs1A, hs1B, srcp, dstp, zerosH)
    hs2A, hs2B = _tc_gcn_finish_project(ss1A, ss1B, hs1A, hs1B, degA, degB, bg1, Wg2)
    ss2A, ss2B = _sc_edge_pass(hs2A, hs2B, srcp, dstp, zerosH)
    return _tc_final(ss2A, ss2B, hs2A, hs2B, degA, degB, bg2, W2, b2, W3, b3)


# speedup vs baseline: 8.1162x; 1.0396x over previous
import functools

import jax
import jax.numpy as jnp
from jax import lax
from jax.experimental import pallas as pl
from jax.experimental.pallas import tpu as pltpu
from jax.experimental.pallas import tpu_sc as plsc

N = 10000
E = 320000
D_IN = 128
H = 256
HH = 128
D_OUT = 64

NC = 2
NS = 16
CH = 64
EPAD = 327680
CPT_EDGE = EPAD // (NS * CH)
CPT_DEG = EPAD // (NC * NS * CH)
GW = 4
N_GROUPS = EPAD // (GW * CH)
GPT_EDGE = CPT_EDGE // GW
GPT_DEG = CPT_DEG // GW
ACC_ROWS = 10240
ZR_PT = ACC_ROWS // NS
DEG_W = 128

BR = 1000
GR = N // BR


def _sc_degree(dstp, ones128, zeros128):
    mesh = plsc.VectorSubcoreMesh(core_axis_name="c", subcore_axis_name="s")

    @functools.partial(
        pl.kernel,
        out_type=(jax.ShapeDtypeStruct((N, DEG_W), jnp.float32),
                  jax.ShapeDtypeStruct((N, DEG_W), jnp.float32)),
        mesh=mesh,
        scratch_types=[
            pltpu.VMEM((CPT_DEG, CH), jnp.int32),
            pltpu.VMEM((CH, DEG_W), jnp.float32),
            pltpu.VMEM((CH, DEG_W), jnp.float32),
            pltpu.VMEM_SHARED((ACC_ROWS, DEG_W), jnp.float32),
            pltpu.SemaphoreType.DMA,
            pltpu.SemaphoreType.DMA,
        ],
        interpret=False,
    )
    def k(dst_ref, ones_ref, zeros_ref, degA, degB,
          didx2, onesbuf, stage, acc, bsem, ssem):
        c = lax.axis_index("c")
        s = lax.axis_index("s")
        pltpu.sync_copy(ones_ref, onesbuf)
        pltpu.sync_copy(zeros_ref, stage)
        for j in range(ZR_PT // CH):
            pltpu.sync_copy(stage, acc.at[pl.ds(s * ZR_PT + j * CH, CH)])
        w = s * NC + c
        bds = [pltpu.async_copy(
                   dst_ref.at[pl.ds(pl.multiple_of(w * (CPT_DEG * CH) + j * CH, 8), CH)],
                   didx2.at[j], bsem)
               for j in range(CPT_DEG)]
        for d in bds:
            d.wait()
        plsc.subcore_barrier()

        def body(g, carry):
            sds = [pltpu.async_copy(onesbuf, acc.at[didx2.at[g * GW + u]],
                                    ssem, add=True)
                   for u in range(GW)]
            for d in sds:
                d.wait()
            return carry

        lax.fori_loop(0, GPT_DEG, body, 0)
        plsc.subcore_barrier()

        def readout(out_ref):
            for j in range(ZR_PT // CH):
                rbase = pl.multiple_of(s * ZR_PT + j * CH, 8)
                full = rbase + CH <= N

                @pl.when(full)
                def _():
                    pltpu.sync_copy(acc.at[pl.ds(rbase, CH)], stage)
                    pltpu.sync_copy(stage, out_ref.at[pl.ds(rbase, CH)])

                @pl.when(jnp.logical_and(jnp.logical_not(full), rbase < N))
                def _():
                    tail = N % CH
                    pltpu.sync_copy(acc.at[pl.ds(rbase, tail)], stage.at[pl.ds(0, tail)])
                    pltpu.sync_copy(stage.at[pl.ds(0, tail)], out_ref.at[pl.ds(rbase, tail)])

        @pl.when(c == 0)
        def _():
            readout(degA)

        @pl.when(c == 1)
        def _():
            readout(degB)

    return k(dstp, ones128, zeros128)


def _sc_edge_pass(hsA, hsB, srcp, dstp, zerosH):
    mesh = plsc.VectorSubcoreMesh(core_axis_name="c", subcore_axis_name="s")

    @functools.partial(
        pl.kernel,
        out_type=(jax.ShapeDtypeStruct((N, HH), jnp.float32),
                  jax.ShapeDtypeStruct((N, HH), jnp.float32)),
        mesh=mesh,
        scratch_types=[
            pltpu.VMEM((GW, CH), jnp.int32),
            pltpu.VMEM((GW, CH), jnp.int32),
            pltpu.VMEM((GW, CH), jnp.int32),
            pltpu.VMEM((GW, CH), jnp.int32),
            pltpu.VMEM((GW, CH, HH), jnp.float32),
            pltpu.VMEM_SHARED((ACC_ROWS, HH), jnp.float32),
            pltpu.SemaphoreType.DMA,
            pltpu.SemaphoreType.DMA,
            pltpu.SemaphoreType.DMA,
            pltpu.SemaphoreType.DMA,
        ],
        interpret=False,
    )
    def k(hsA_ref, hsB_ref, src_ref, dst_ref, zeros_ref, ssA, ssB,
          sidxA, didxA, sidxB, didxB, rows, acc, bsemA, bsemB, gsem, ssem):
        c = lax.axis_index("c")
        s = lax.axis_index("s")
        stage = rows.at[0]
        pltpu.sync_copy(zeros_ref, stage)
        for j in range(ZR_PT // CH):
            pltpu.sync_copy(stage, acc.at[pl.ds(s * ZR_PT + j * CH, CH)])
        plsc.subcore_barrier()

        def fire_idx(bank_s, bank_d, g, sem):
            for u in range(GW):
                base = pl.multiple_of(
                    s * (CPT_EDGE * CH) + (g * GW + u) * CH, 8)
                pltpu.async_copy(src_ref.at[pl.ds(base, CH)], bank_s.at[u], sem)
                pltpu.async_copy(dst_ref.at[pl.ds(base, CH)], bank_d.at[u], sem)

        def drain_idx(bank_s, bank_d, sem):
            for u in range(GW):
                pltpu.make_async_copy(src_ref.at[pl.ds(0, CH)],
                                      bank_s.at[u], sem).wait()
                pltpu.make_async_copy(dst_ref.at[pl.ds(0, CH)],
                                      bank_d.at[u], sem).wait()

        def run(hs_ref, ss_ref):
            def data_phase(bank_s, bank_d):
                gds = [pltpu.async_copy(hs_ref.at[bank_s.at[u]],
                                        rows.at[u], gsem)
                       for u in range(GW)]
                sds = []
                for u in range(GW):
                    gds[u].wait()
                    sds.append(pltpu.async_copy(rows.at[u],
                                                acc.at[bank_d.at[u]],
                                                ssem, add=True))
                for d in sds:
                    d.wait()

            fire_idx(sidxA, didxA, 0, bsemA)

            def body(r, carry):
                gB = 2 * r + 1
                fire_idx(sidxB, didxB, gB, bsemB)
                drain_idx(sidxA, didxA, bsemA)
                data_phase(sidxA, didxA)

                @pl.when(2 * r + 2 < GPT_EDGE)
                def _():
                    fire_idx(sidxA, didxA, 2 * r + 2, bsemA)

                drain_idx(sidxB, didxB, bsemB)
                data_phase(sidxB, didxB)
                return carry

            lax.fori_loop(0, GPT_EDGE // 2, body, 0)
            plsc.subcore_barrier()
            for j in range(ZR_PT // CH):
                rbase = pl.multiple_of(s * ZR_PT + j * CH, 8)
                full = rbase + CH <= N

                @pl.when(full)
                def _():
                    pltpu.sync_copy(acc.at[pl.ds(rbase, CH)], stage)
                    pltpu.sync_copy(stage, ss_ref.at[pl.ds(rbase, CH)])

                @pl.when(jnp.logical_and(jnp.logical_not(full), rbase < N))
                def _():
                    tail = N % CH
                    pltpu.sync_copy(acc.at[pl.ds(rbase, tail)],
                                    stage.at[pl.ds(0, tail)])
                    pltpu.sync_copy(stage.at[pl.ds(0, tail)],
                                    ss_ref.at[pl.ds(rbase, tail)])

        @pl.when(c == 0)
        def _():
            run(hsA_ref, ssA)

        @pl.when(c == 1)
        def _():
            run(hsB_ref, ssB)

    return k(hsA, hsB, srcp, dstp, zerosH)


def _dinv_block(dA_ref, dB_ref):
    deg = dA_ref[:, :1] + dB_ref[:, :1] + 1.0
    return lax.rsqrt(deg)


def _tc_in_mlp(x, W1, b1):
    def body(x_ref, w_ref, b_ref, o_ref):
        o_ref[...] = jnp.maximum(
            jnp.dot(x_ref[...], w_ref[...], preferred_element_type=jnp.float32)
            + b_ref[...], 0.0)

    return pl.pallas_call(
        body,
        grid=(GR,),
        in_specs=[pl.BlockSpec((BR, D_IN), lambda i: (i, 0)),
                  pl.BlockSpec((D_IN, H), lambda i: (0, 0)),
                  pl.BlockSpec((1, H), lambda i: (0, 0))],
        out_specs=pl.BlockSpec((BR, H), lambda i: (i, 0)),
        out_shape=jax.ShapeDtypeStruct((N, H), jnp.float32),
        interpret=False,
    )(x, W1, b1.reshape(1, H))


def _tc_scale_project(h, Wg, degA, degB):
    def body(h_ref, w_ref, dA_ref, dB_ref, oA, oB):
        dinv = _dinv_block(dA_ref, dB_ref)
        hw = jnp.dot(h_ref[...], w_ref[...], preferred_element_type=jnp.float32)
        hs = hw * dinv
        oA[...] = hs[:, :HH]
        oB[...] = hs[:, HH:]

    return pl.pallas_call(
        body,
        grid=(GR,),
        in_specs=[pl.BlockSpec((BR, H), lambda i: (i, 0)),
                  pl.BlockSpec((H, H), lambda i: (0, 0)),
                  pl.BlockSpec((BR, DEG_W), lambda i: (i, 0)),
                  pl.BlockSpec((BR, DEG_W), lambda i: (i, 0))],
        out_specs=[pl.BlockSpec((BR, HH), lambda i: (i, 0)),
                   pl.BlockSpec((BR, HH), lambda i: (i, 0))],
        out_shape=[jax.ShapeDtypeStruct((N, HH), jnp.float32),
                   jax.ShapeDtypeStruct((N, HH), jnp.float32)],
        interpret=False,
    )(h, Wg, degA, degB)


def _tc_gcn_finish_project(ssA, ssB, hsA, hsB, degA, degB, bg, Wg2):
    def body(ssA_ref, ssB_ref, hsA_ref, hsB_ref, dA_ref, dB_ref, bg_ref,
             w_ref, oA, oB):
        dinv = _dinv_block(dA_ref, dB_ref)
        h = jnp.concatenate(
            [ssA_ref[...] + hsA_ref[...], ssB_ref[...] + hsB_ref[...]], axis=1)
        h = jnp.maximum(h * dinv + bg_ref[...], 0.0)
        hw = jnp.dot(h, w_ref[...], preferred_element_type=jnp.float32)
        hs = hw * dinv
        oA[...] = hs[:, :HH]
        oB[...] = hs[:, HH:]

    return pl.pallas_call(
        body,
        grid=(GR,),
        in_specs=[pl.BlockSpec((BR, HH), lambda i: (i, 0)),
                  pl.BlockSpec((BR, HH), lambda i: (i, 0)),
                  pl.BlockSpec((BR, HH), lambda i: (i, 0)),
                  pl.BlockSpec((BR, HH), lambda i: (i, 0)),
                  pl.BlockSpec((BR, DEG_W), lambda i: (i, 0)),
                  pl.BlockSpec((BR, DEG_W), lambda i: (i, 0)),
                  pl.BlockSpec((1, H), lambda i: (0, 0)),
                  pl.BlockSpec((H, H), lambda i: (0, 0))],
        out_specs=[pl.BlockSpec((BR, HH), lambda i: (i, 0)),
                   pl.BlockSpec((BR, HH), lambda i: (i, 0))],
        out_shape=[jax.ShapeDtypeStruct((N, HH), jnp.float32),
                   jax.ShapeDtypeStruct((N, HH), jnp.float32)],
        interpret=False,
    )(ssA, ssB, hsA, hsB, degA, degB, bg.reshape(1, H), Wg2)


def _tc_final(ssA, ssB, hsA, hsB, degA, degB, bg2, W2, b2, W3, b3):
    def body(ssA_ref, ssB_ref, hsA_ref, hsB_ref, dA_ref, dB_ref, bg2_ref,
             w2_ref, b2_ref, w3_ref, b3_ref, o_ref):
        dinv = _dinv_block(dA_ref, dB_ref)
        h = jnp.concatenate(
            [ssA_ref[...] + hsA_ref[...], ssB_ref[...] + hsB_ref[...]], axis=1)
        h = jnp.maximum(h * dinv + bg2_ref[...], 0.0)
        h = jnp.maximum(
            jnp.dot(h, w2_ref[...], preferred_element_type=jnp.float32)
            + b2_ref[...], 0.0)
        z = jnp.dot(h, w3_ref[...], preferred_element_type=jnp.float32) + b3_ref[...]
        z = z - jnp.max(z, axis=1, keepdims=True)
        ez = jnp.exp(z)
        o_ref[...] = ez / jnp.sum(ez, axis=1, keepdims=True)

    return pl.pallas_call(
        body,
        grid=(GR,),
        in_specs=[pl.BlockSpec((BR, HH), lambda i: (i, 0)),
                  pl.BlockSpec((BR, HH), lambda i: (i, 0)),
                  pl.BlockSpec((BR, HH), lambda i: (i, 0)),
                  pl.BlockSpec((BR, HH), lambda i: (i, 0)),
                  pl.BlockSpec((BR, DEG_W), lambda i: (i, 0)),
                  pl.BlockSpec((BR, DEG_W), lambda i: (i, 0)),
                  pl.BlockSpec((1, H), lambda i: (0, 0)),
                  pl.BlockSpec((H, H), lambda i: (0, 0)),
                  pl.BlockSpec((1, H), lambda i: (0, 0)),
                  pl.BlockSpec((H, D_OUT), lambda i: (0, 0)),
                  pl.BlockSpec((1, D_OUT), lambda i: (0, 0))],
        out_specs=pl.BlockSpec((BR, D_OUT), lambda i: (i, 0)),
        out_shape=jax.ShapeDtypeStruct((N, D_OUT), jnp.float32),
        interpret=False,
    )(ssA, ssB, hsA, hsB, degA, degB, bg2.reshape(1, H), W2,
      b2.reshape(1, H), W3, b3.reshape(1, D_OUT))


def kernel(x, edge_index, W1, b1, Wg1, bg1, Wg2, bg2, W2, b2, W3, b3):
    src = edge_index[0].astype(jnp.int32)
    dst = edge_index[1].astype(jnp.int32)
    srcp = jnp.concatenate([src, jnp.zeros((EPAD - E,), jnp.int32)])
    dstp = jnp.concatenate([dst, jnp.full((EPAD - E,), N, jnp.int32)])
    ones128 = jnp.ones((CH, DEG_W), jnp.float32)
    zeros128 = jnp.zeros((CH, DEG_W), jnp.float32)
    zerosH = jnp.zeros((CH, HH), jnp.float32)

    degA, degB = _sc_degree(dstp, ones128, zeros128)
    h0 = _tc_in_mlp(x, W1, b1)
    hs1A, hs1B = _tc_scale_project(h0, Wg1, degA, degB)
    ss1A, ss1B = _sc_edge_pass(hs1A, hs1B, srcp, dstp, zerosH)
    hs2A, hs2B = _tc_gcn_finish_project(ss1A, ss1B, hs1A, hs1B, degA, degB, bg1, Wg2)
    ss2A, ss2B = _sc_edge_pass(hs2A, hs2B, srcp, dstp, zerosH)
    return _tc_final(ss2A, ss2B, hs2A, hs2B, degA, degB, bg2, W2, b2, W3, b3)


# speedup vs baseline: 8.8379x; 1.0889x over previous
import functools

import jax
import jax.numpy as jnp
from jax import lax
from jax.experimental import pallas as pl
from jax.experimental.pallas import tpu as pltpu
from jax.experimental.pallas import tpu_sc as plsc

N = 10000
E = 320000
D_IN = 128
H = 256
HH = 128
D_OUT = 64

NC = 2
NS = 16
CH = 32
EPAD = 327680
CPT_EDGE = EPAD // (NS * CH)
CPT_DEG = EPAD // (NC * NS * CH)
GW = 8
N_GROUPS = EPAD // (GW * CH)
GPT_EDGE = CPT_EDGE // GW
GPT_DEG = CPT_DEG // GW
ACC_ROWS = 10240
ZR_PT = ACC_ROWS // NS
DEG_W = 128

BR = 1000
GR = N // BR


def _sc_degree(dstp, ones128, zeros128):
    mesh = plsc.VectorSubcoreMesh(core_axis_name="c", subcore_axis_name="s")

    @functools.partial(
        pl.kernel,
        out_type=(jax.ShapeDtypeStruct((N, DEG_W), jnp.float32),
                  jax.ShapeDtypeStruct((N, DEG_W), jnp.float32)),
        mesh=mesh,
        scratch_types=[
            pltpu.VMEM((CPT_DEG, CH), jnp.int32),
            pltpu.VMEM((CH, DEG_W), jnp.float32),
            pltpu.VMEM((CH, DEG_W), jnp.float32),
            pltpu.VMEM_SHARED((ACC_ROWS, DEG_W), jnp.float32),
            pltpu.SemaphoreType.DMA,
            pltpu.SemaphoreType.DMA,
        ],
        interpret=False,
    )
    def k(dst_ref, ones_ref, zeros_ref, degA, degB,
          didx2, onesbuf, stage, acc, bsem, ssem):
        c = lax.axis_index("c")
        s = lax.axis_index("s")
        pltpu.sync_copy(ones_ref, onesbuf)
        pltpu.sync_copy(zeros_ref, stage)
        for j in range(ZR_PT // CH):
            pltpu.sync_copy(stage, acc.at[pl.ds(s * ZR_PT + j * CH, CH)])
        w = s * NC + c
        bds = [pltpu.async_copy(
                   dst_ref.at[pl.ds(pl.multiple_of(w * (CPT_DEG * CH) + j * CH, 8), CH)],
                   didx2.at[j], bsem)
               for j in range(CPT_DEG)]
        for d in bds:
            d.wait()
        plsc.subcore_barrier()

        def body(g, carry):
            sds = [pltpu.async_copy(onesbuf, acc.at[didx2.at[g * GW + u]],
                                    ssem, add=True)
                   for u in range(GW)]
            for d in sds:
                d.wait()
            return carry

        lax.fori_loop(0, GPT_DEG, body, 0)
        plsc.subcore_barrier()

        def readout(out_ref):
            for j in range(ZR_PT // CH):
                rbase = pl.multiple_of(s * ZR_PT + j * CH, 8)
                full = rbase + CH <= N

                @pl.when(full)
                def _():
                    pltpu.sync_copy(acc.at[pl.ds(rbase, CH)], stage)
                    pltpu.sync_copy(stage, out_ref.at[pl.ds(rbase, CH)])

                @pl.when(jnp.logical_and(jnp.logical_not(full), rbase < N))
                def _():
                    tail = N % CH
                    pltpu.sync_copy(acc.at[pl.ds(rbase, tail)], stage.at[pl.ds(0, tail)])
                    pltpu.sync_copy(stage.at[pl.ds(0, tail)], out_ref.at[pl.ds(rbase, tail)])

        @pl.when(c == 0)
        def _():
            readout(degA)

        @pl.when(c == 1)
        def _():
            readout(degB)

    return k(dstp, ones128, zeros128)


def _sc_edge_pass(hsA, hsB, srcp, dstp, zerosH):
    mesh = plsc.VectorSubcoreMesh(core_axis_name="c", subcore_axis_name="s")

    @functools.partial(
        pl.kernel,
        out_type=(jax.ShapeDtypeStruct((N, HH), jnp.float32),
                  jax.ShapeDtypeStruct((N, HH), jnp.float32)),
        mesh=mesh,
        scratch_types=[
            pltpu.VMEM((GW, CH), jnp.int32),
            pltpu.VMEM((GW, CH), jnp.int32),
            pltpu.VMEM((GW, CH), jnp.int32),
            pltpu.VMEM((GW, CH), jnp.int32),
            pltpu.VMEM((GW, CH, HH), jnp.float32),
            pltpu.VMEM_SHARED((ACC_ROWS, HH), jnp.float32),
            pltpu.SemaphoreType.DMA,
            pltpu.SemaphoreType.DMA,
            pltpu.SemaphoreType.DMA,
            pltpu.SemaphoreType.DMA,
        ],
        interpret=False,
    )
    def k(hsA_ref, hsB_ref, src_ref, dst_ref, zeros_ref, ssA, ssB,
          sidxA, didxA, sidxB, didxB, rows, acc, bsemA, bsemB, gsem, ssem):
        c = lax.axis_index("c")
        s = lax.axis_index("s")
        stage = rows.at[0]
        pltpu.sync_copy(zeros_ref, stage)
        for j in range(ZR_PT // CH):
            pltpu.sync_copy(stage, acc.at[pl.ds(s * ZR_PT + j * CH, CH)])
        plsc.subcore_barrier()

        def fire_idx(bank_s, bank_d, g, sem):
            for u in range(GW):
                base = pl.multiple_of(
                    s * (CPT_EDGE * CH) + (g * GW + u) * CH, 8)
                pltpu.async_copy(src_ref.at[pl.ds(base, CH)], bank_s.at[u], sem)
                pltpu.async_copy(dst_ref.at[pl.ds(base, CH)], bank_d.at[u], sem)

        def drain_idx(bank_s, bank_d, sem):
            for u in range(GW):
                pltpu.make_async_copy(src_ref.at[pl.ds(0, CH)],
                                      bank_s.at[u], sem).wait()
                pltpu.make_async_copy(dst_ref.at[pl.ds(0, CH)],
                                      bank_d.at[u], sem).wait()

        def run(hs_ref, ss_ref):
            def data_phase(bank_s, bank_d):
                gds = [pltpu.async_copy(hs_ref.at[bank_s.at[u]],
                                        rows.at[u], gsem)
                       for u in range(GW)]
                sds = []
                for u in range(GW):
                    gds[u].wait()
                    sds.append(pltpu.async_copy(rows.at[u],
                                                acc.at[bank_d.at[u]],
                                                ssem, add=True))
                for d in sds:
                    d.wait()

            fire_idx(sidxA, didxA, 0, bsemA)

            def body(r, carry):
                gB = 2 * r + 1
                fire_idx(sidxB, didxB, gB, bsemB)
                drain_idx(sidxA, didxA, bsemA)
                data_phase(sidxA, didxA)

                @pl.when(2 * r + 2 < GPT_EDGE)
                def _():
                    fire_idx(sidxA, didxA, 2 * r + 2, bsemA)

                drain_idx(sidxB, didxB, bsemB)
                data_phase(sidxB, didxB)
                return carry

            lax.fori_loop(0, GPT_EDGE // 2, body, 0)
            plsc.subcore_barrier()
            for j in range(ZR_PT // CH):
                rbase = pl.multiple_of(s * ZR_PT + j * CH, 8)
                full = rbase + CH <= N

                @pl.when(full)
                def _():
                    pltpu.sync_copy(acc.at[pl.ds(rbase, CH)], stage)
                    pltpu.sync_copy(stage, ss_ref.at[pl.ds(rbase, CH)])

                @pl.when(jnp.logical_and(jnp.logical_not(full), rbase < N))
                def _():
                    tail = N % CH
                    pltpu.sync_copy(acc.at[pl.ds(rbase, tail)],
                                    stage.at[pl.ds(0, tail)])
                    pltpu.sync_copy(stage.at[pl.ds(0, tail)],
                                    ss_ref.at[pl.ds(rbase, tail)])

        @pl.when(c == 0)
        def _():
            run(hsA_ref, ssA)

        @pl.when(c == 1)
        def _():
            run(hsB_ref, ssB)

    return k(hsA, hsB, srcp, dstp, zerosH)


def _dinv_block(dA_ref, dB_ref):
    deg = dA_ref[:, :1] + dB_ref[:, :1] + 1.0
    return lax.rsqrt(deg)


def _tc_in_mlp(x, W1, b1):
    def body(x_ref, w_ref, b_ref, o_ref):
        o_ref[...] = jnp.maximum(
            jnp.dot(x_ref[...], w_ref[...], preferred_element_type=jnp.float32)
            + b_ref[...], 0.0)

    return pl.pallas_call(
        body,
        grid=(GR,),
        in_specs=[pl.BlockSpec((BR, D_IN), lambda i: (i, 0)),
                  pl.BlockSpec((D_IN, H), lambda i: (0, 0)),
                  pl.BlockSpec((1, H), lambda i: (0, 0))],
        out_specs=pl.BlockSpec((BR, H), lambda i: (i, 0)),
        out_shape=jax.ShapeDtypeStruct((N, H), jnp.float32),
        interpret=False,
    )(x, W1, b1.reshape(1, H))


def _tc_scale_project(h, Wg, degA, degB):
    def body(h_ref, w_ref, dA_ref, dB_ref, oA, oB):
        dinv = _dinv_block(dA_ref, dB_ref)
        hw = jnp.dot(h_ref[...], w_ref[...], preferred_element_type=jnp.float32)
        hs = hw * dinv
        oA[...] = hs[:, :HH]
        oB[...] = hs[:, HH:]

    return pl.pallas_call(
        body,
        grid=(GR,),
        in_specs=[pl.BlockSpec((BR, H), lambda i: (i, 0)),
                  pl.BlockSpec((H, H), lambda i: (0, 0)),
                  pl.BlockSpec((BR, DEG_W), lambda i: (i, 0)),
                  pl.BlockSpec((BR, DEG_W), lambda i: (i, 0))],
        out_specs=[pl.BlockSpec((BR, HH), lambda i: (i, 0)),
                   pl.BlockSpec((BR, HH), lambda i: (i, 0))],
        out_shape=[jax.ShapeDtypeStruct((N, HH), jnp.float32),
                   jax.ShapeDtypeStruct((N, HH), jnp.float32)],
        interpret=False,
    )(h, Wg, degA, degB)


def _tc_gcn_finish_project(ssA, ssB, hsA, hsB, degA, degB, bg, Wg2):
    def body(ssA_ref, ssB_ref, hsA_ref, hsB_ref, dA_ref, dB_ref, bg_ref,
             w_ref, oA, oB):
        dinv = _dinv_block(dA_ref, dB_ref)
        h = jnp.concatenate(
            [ssA_ref[...] + hsA_ref[...], ssB_ref[...] + hsB_ref[...]], axis=1)
        h = jnp.maximum(h * dinv + bg_ref[...], 0.0)
        hw = jnp.dot(h, w_ref[...], preferred_element_type=jnp.float32)
        hs = hw * dinv
        oA[...] = hs[:, :HH]
        oB[...] = hs[:, HH:]

    return pl.pallas_call(
        body,
        grid=(GR,),
        in_specs=[pl.BlockSpec((BR, HH), lambda i: (i, 0)),
                  pl.BlockSpec((BR, HH), lambda i: (i, 0)),
                  pl.BlockSpec((BR, HH), lambda i: (i, 0)),
                  pl.BlockSpec((BR, HH), lambda i: (i, 0)),
                  pl.BlockSpec((BR, DEG_W), lambda i: (i, 0)),
                  pl.BlockSpec((BR, DEG_W), lambda i: (i, 0)),
                  pl.BlockSpec((1, H), lambda i: (0, 0)),
                  pl.BlockSpec((H, H), lambda i: (0, 0))],
        out_specs=[pl.BlockSpec((BR, HH), lambda i: (i, 0)),
                   pl.BlockSpec((BR, HH), lambda i: (i, 0))],
        out_shape=[jax.ShapeDtypeStruct((N, HH), jnp.float32),
                   jax.ShapeDtypeStruct((N, HH), jnp.float32)],
        interpret=False,
    )(ssA, ssB, hsA, hsB, degA, degB, bg.reshape(1, H), Wg2)


def _tc_final(ssA, ssB, hsA, hsB, degA, degB, bg2, W2, b2, W3, b3):
    def body(ssA_ref, ssB_ref, hsA_ref, hsB_ref, dA_ref, dB_ref, bg2_ref,
             w2_ref, b2_ref, w3_ref, b3_ref, o_ref):
        dinv = _dinv_block(dA_ref, dB_ref)
        h = jnp.concatenate(
            [ssA_ref[...] + hsA_ref[...], ssB_ref[...] + hsB_ref[...]], axis=1)
        h = jnp.maximum(h * dinv + bg2_ref[...], 0.0)
        h = jnp.maximum(
            jnp.dot(h, w2_ref[...], preferred_element_type=jnp.float32)
            + b2_ref[...], 0.0)
        z = jnp.dot(h, w3_ref[...], preferred_element_type=jnp.float32) + b3_ref[...]
        z = z - jnp.max(z, axis=1, keepdims=True)
        ez = jnp.exp(z)
        o_ref[...] = ez / jnp.sum(ez, axis=1, keepdims=True)

    return pl.pallas_call(
        body,
        grid=(GR,),
        in_specs=[pl.BlockSpec((BR, HH), lambda i: (i, 0)),
                  pl.BlockSpec((BR, HH), lambda i: (i, 0)),
                  pl.BlockSpec((BR, HH), lambda i: (i, 0)),
                  pl.BlockSpec((BR, HH), lambda i: (i, 0)),
                  pl.BlockSpec((BR, DEG_W), lambda i: (i, 0)),
                  pl.BlockSpec((BR, DEG_W), lambda i: (i, 0)),
                  pl.BlockSpec((1, H), lambda i: (0, 0)),
                  pl.BlockSpec((H, H), lambda i: (0, 0)),
                  pl.BlockSpec((1, H), lambda i: (0, 0)),
                  pl.BlockSpec((H, D_OUT), lambda i: (0, 0)),
                  pl.BlockSpec((1, D_OUT), lambda i: (0, 0))],
        out_specs=pl.BlockSpec((BR, D_OUT), lambda i: (i, 0)),
        out_shape=jax.ShapeDtypeStruct((N, D_OUT), jnp.float32),
        interpret=False,
    )(ssA, ssB, hsA, hsB, degA, degB, bg2.reshape(1, H), W2,
      b2.reshape(1, H), W3, b3.reshape(1, D_OUT))


def kernel(x, edge_index, W1, b1, Wg1, bg1, Wg2, bg2, W2, b2, W3, b3):
    src = edge_index[0].astype(jnp.int32)
    dst = edge_index[1].astype(jnp.int32)
    srcp = jnp.concatenate([src, jnp.zeros((EPAD - E,), jnp.int32)])
    dstp = jnp.concatenate([dst, jnp.full((EPAD - E,), N, jnp.int32)])
    ones128 = jnp.ones((CH, DEG_W), jnp.float32)
    zeros128 = jnp.zeros((CH, DEG_W), jnp.float32)
    zerosH = jnp.zeros((CH, HH), jnp.float32)

    degA, degB = _sc_degree(dstp, ones128, zeros128)
    h0 = _tc_in_mlp(x, W1, b1)
    hs1A, hs1B = _tc_scale_project(h0, Wg1, degA, degB)
    ss1A, ss1B = _sc_edge_pass(hs1A, hs1B, srcp, dstp, zerosH)
    hs2A, hs2B = _tc_gcn_finish_project(ss1A, ss1B, hs1A, hs1B, degA, degB, bg1, Wg2)
    ss2A, ss2B = _sc_edge_pass(hs2A, hs2B, srcp, dstp, zerosH)
    return _tc_final(ss2A, ss2B, hs2A, hs2B, degA, degB, bg2, W2, b2, W3, b3)
